# baseline (speedup 1.0000x reference)
import jax
import jax.numpy as jnp
from jax.experimental import pallas as pl
from jax.experimental.pallas import tpu as pltpu
from jax.experimental.pallas import tpu_sc as plsc

OBS_DIM = 64
N_CODE_EACH = 512
CODE_DIM = 64
BATCH = 1024
HIDDEN = 256
N_CODE_TOTAL = OBS_DIM * N_CODE_EACH
REP_DIM = OBS_DIM * CODE_DIM

A_CHUNKS = 2
G_CHUNKS = 4
AB = BATCH // A_CHUNKS
GB = BATCH // G_CHUNKS


def _argmin_body(emb_ref, obsT_ref, encwT_ref, encbT_ref, fidx2_ref, par_ref):
    j = pl.program_id(0)
    Wt = emb_ref[...]
    ohj = jax.lax.broadcasted_iota(jnp.int32, (CODE_DIM, OBS_DIM), 1) == j
    ewc = jnp.sum(jnp.where(ohj, encwT_ref[...], 0.0), axis=1, keepdims=True)
    ebc = jnp.sum(jnp.where(ohj, encbT_ref[...], 0.0), axis=1, keepdims=True)
    ob = obsT_ref[pl.ds(j, 1), :]
    zT = ob * ewc + ebc
    cross = jnp.dot(Wt, zT, preferred_element_type=jnp.float32)
    z2 = jnp.sum(zT * zT, axis=0, keepdims=True)
    w2 = jnp.sum(Wt * Wt, axis=1, keepdims=True)
    dists = (z2 + w2) - 2.0 * cross
    m = jnp.min(dists, axis=0, keepdims=True)
    kio = jax.lax.broadcasted_iota(jnp.int32, dists.shape, 0)
    cand = jnp.where(dists == m, kio, N_CODE_EACH)
    idx = jnp.min(cand, axis=0, keepdims=True)
    fidx2_ref[pl.ds(j, 1), :] = (idx >> 1) + j * (N_CODE_EACH // 2)
    par_ref[pl.ds(j, 1), :] = idx & 1


def _ze_body(obs_ref, encwT_ref, encbT_ref, ze_ref):
    ze_ref[...] = (obs_ref[...][:, None, :] * encwT_ref[...][None, :, :]
                   + encbT_ref[...][None, :, :])


def _decoder_body(q2_ref, par_ref, w1_ref, b1_ref, w2_ref, b2_ref,
                  recon_ref, emb_ref):
    q2 = q2_ref[...]
    sel = jnp.where(par_ref[...] == 0,
                    q2[:, :, :CODE_DIM], q2[:, :, CODE_DIM:])
    emb_ref[...] = jnp.swapaxes(sel, 1, 2)
    qf = sel.reshape(sel.shape[0], REP_DIM)
    h = jnp.dot(qf, w1_ref[...],
                preferred_element_type=jnp.float32) + b1_ref[...]
    h = jnp.maximum(h, 0.0)
    recon_ref[...] = jnp.dot(h, w2_ref[...],
                             preferred_element_type=jnp.float32) + b2_ref[...]


def _sc_gather(table, fidx2):
    n_idx = fidx2.shape[1]
    window = 256

    @pl.kernel(
        out_type=jax.ShapeDtypeStruct((n_idx, 2 * CODE_DIM), table.dtype),
        mesh=plsc.VectorSubcoreMesh(core_axis_name="core",
                                    subcore_axis_name="subcore"),
    )
    def kern(x_hbm, i_hbm, o_hbm):
        def body(i_vmem, o_vmem):
            pltpu.sync_copy(x_hbm.at[i_vmem.at[0]], o_vmem)

        pltpu.emit_pipeline(
            body,
            grid=(n_idx // window,),
            in_specs=[pl.BlockSpec((1, window), index_map=lambda i: (0, i))],
            out_specs=[pl.BlockSpec((window, 2 * CODE_DIM),
                                    index_map=lambda i: (i, 0))],
            core_axis_name=("core", "subcore"),
            dimension_semantics=(pltpu.PARALLEL,),
        )(i_hbm, o_hbm)

    return kern(table, fidx2)


def _argmin_chunk(embT, obsT_c, encwT, encbT):
    nb = obsT_c.shape[1]
    return pl.pallas_call(
        _argmin_body,
        grid=(OBS_DIM,),
        in_specs=[
            pl.BlockSpec((N_CODE_EACH, CODE_DIM), lambda j: (j, 0)),
            pl.BlockSpec((OBS_DIM, nb), lambda j: (0, 0)),
            pl.BlockSpec((CODE_DIM, OBS_DIM), lambda j: (0, 0)),
            pl.BlockSpec((CODE_DIM, OBS_DIM), lambda j: (0, 0)),
        ],
        out_specs=[
            pl.BlockSpec((OBS_DIM, nb), lambda j: (0, 0)),
            pl.BlockSpec((OBS_DIM, nb), lambda j: (0, 0)),
        ],
        out_shape=[
            jax.ShapeDtypeStruct((OBS_DIM, nb), jnp.int32),
            jax.ShapeDtypeStruct((OBS_DIM, nb), jnp.int32),
        ],
    )(embT, obsT_c, encwT, encbT)


def _decoder_chunk(q2v_c, par3_c, dec_w1, b1r, dec_w2, b2r):
    gb = q2v_c.shape[0]
    return pl.pallas_call(
        _decoder_body,
        grid=(1,),
        in_specs=[
            pl.BlockSpec((gb, OBS_DIM, 2 * CODE_DIM), lambda i: (0, 0, 0)),
            pl.BlockSpec((gb, OBS_DIM, 1), lambda i: (0, 0, 0)),
            pl.BlockSpec((REP_DIM, HIDDEN), lambda i: (0, 0)),
            pl.BlockSpec((1, HIDDEN), lambda i: (0, 0)),
            pl.BlockSpec((HIDDEN, OBS_DIM), lambda i: (0, 0)),
            pl.BlockSpec((1, OBS_DIM), lambda i: (0, 0)),
        ],
        out_specs=[
            pl.BlockSpec((gb, OBS_DIM), lambda i: (0, 0)),
            pl.BlockSpec((gb, CODE_DIM, OBS_DIM), lambda i: (0, 0, 0)),
        ],
        out_shape=[
            jax.ShapeDtypeStruct((gb, OBS_DIM), jnp.float32),
            jax.ShapeDtypeStruct((gb, CODE_DIM, OBS_DIM), jnp.float32),
        ],
    )(q2v_c, par3_c, dec_w1, b1r, dec_w2, b2r)


def kernel(obs, enc_w, enc_b, emb_weight, dec_w1, dec_b1, dec_w2, dec_b2):
    embT = emb_weight.T
    pair_table = embT.reshape(N_CODE_TOTAL // 2, 2 * CODE_DIM)
    encwT = enc_w.T
    encbT = enc_b.T
    b1r = dec_b1.reshape(1, HIDDEN)
    b2r = dec_b2.reshape(1, OBS_DIM)

    fidx2_parts, par_parts = [], []
    for c in range(A_CHUNKS):
        obsT_c = jax.lax.slice(obs, (c * AB, 0), ((c + 1) * AB, OBS_DIM)).T
        fidx2T_c, parT_c = _argmin_chunk(embT, obsT_c, encwT, encbT)
        fidx2_parts.append(fidx2T_c.T.reshape(1, AB * OBS_DIM))
        par_parts.append(parT_c.T.reshape(AB, OBS_DIM, 1))

    ze = pl.pallas_call(
        _ze_body,
        grid=(BATCH // 256,),
        in_specs=[
            pl.BlockSpec((256, OBS_DIM), lambda i: (i, 0)),
            pl.BlockSpec((CODE_DIM, OBS_DIM), lambda i: (0, 0)),
            pl.BlockSpec((CODE_DIM, OBS_DIM), lambda i: (0, 0)),
        ],
        out_specs=pl.BlockSpec((256, CODE_DIM, OBS_DIM), lambda i: (i, 0, 0)),
        out_shape=jax.ShapeDtypeStruct((BATCH, CODE_DIM, OBS_DIM),
                                       jnp.float32),
    )(obs, encwT, encbT)

    g_per_a = G_CHUNKS // A_CHUNKS
    recon_parts, emb_parts = [], []
    for c in range(A_CHUNKS):
        for g in range(g_per_a):
            fidx2_g = jax.lax.slice(
                fidx2_parts[c], (0, g * GB * OBS_DIM),
                (1, (g + 1) * GB * OBS_DIM))
            par3_g = jax.lax.slice(
                par_parts[c], (g * GB, 0, 0), ((g + 1) * GB, OBS_DIM, 1))
            q2_g = _sc_gather(pair_table, fidx2_g)
            q2v_g = q2_g.reshape(GB, OBS_DIM, 2 * CODE_DIM)
            recon_g, emb_g = _decoder_chunk(q2v_g, par3_g, dec_w1, b1r,
                                            dec_w2, b2r)
            recon_parts.append(recon_g)
            emb_parts.append(emb_g)

    recon = jnp.concatenate(recon_parts, axis=0)
    emb = jnp.concatenate(emb_parts, axis=0)
    return recon, ze, emb


# speedup vs baseline: 1.3921x; 1.3921x over previous
import jax
import jax.numpy as jnp
from jax.experimental import pallas as pl
from jax.experimental.pallas import tpu as pltpu
from jax.experimental.pallas import tpu_sc as plsc

OBS_DIM = 64
N_CODE_EACH = 512
CODE_DIM = 64
BATCH = 1024
HIDDEN = 256
N_CODE_TOTAL = OBS_DIM * N_CODE_EACH
REP_DIM = OBS_DIM * CODE_DIM


def _argmin_body(emb_ref, obsT_ref, encwT_ref, encbT_ref,
                 fidx2_ref, par_ref, pair_ref):
    j = pl.program_id(0)
    Wt = emb_ref[...].T
    pair_ref[...] = jnp.concatenate(
        [Wt[:N_CODE_EACH // 2, :], Wt[N_CODE_EACH // 2:, :]], axis=1)
    ohj = jax.lax.broadcasted_iota(jnp.int32, (CODE_DIM, OBS_DIM), 1) == j
    ewc = jnp.sum(jnp.where(ohj, encwT_ref[...], 0.0), axis=1, keepdims=True)
    ebc = jnp.sum(jnp.where(ohj, encbT_ref[...], 0.0), axis=1, keepdims=True)
    ob = obsT_ref[pl.ds(j, 1), :]
    zT = ob * ewc + ebc
    cross = jnp.dot(Wt, zT, preferred_element_type=jnp.float32)
    z2 = jnp.sum(zT * zT, axis=0, keepdims=True)
    w2 = jnp.sum(Wt * Wt, axis=1, keepdims=True)
    dists = (z2 + w2) - 2.0 * cross
    m = jnp.min(dists, axis=0, keepdims=True)
    kio = jax.lax.broadcasted_iota(jnp.int32, dists.shape, 0)
    cand = jnp.where(dists == m, kio, N_CODE_EACH)
    idx = jnp.min(cand, axis=0, keepdims=True)
    fidx2_ref[pl.ds(j, 1), :] = (idx & (N_CODE_EACH // 2 - 1)) + j * (
        N_CODE_EACH // 2)
    par_ref[pl.ds(j, 1), :] = idx >> 8


def _decoder_body(q2_ref, par_ref, obs_ref, encwT_ref, encbT_ref,
                  w1_ref, b1_ref, w2_ref, b2_ref,
                  recon_ref, ze_ref, emb_ref):
    q2 = q2_ref[...]
    par3 = par_ref[...][:, :, None]
    sel = jnp.where(par3 == 0,
                    q2[:, :, :CODE_DIM], q2[:, :, CODE_DIM:])
    emb_ref[...] = jnp.swapaxes(sel, 1, 2)
    qf = sel.reshape(sel.shape[0], REP_DIM)
    h = jnp.dot(qf, w1_ref[...],
                preferred_element_type=jnp.float32) + b1_ref[...]
    h = jnp.maximum(h, 0.0)
    recon_ref[...] = jnp.dot(h, w2_ref[...],
                             preferred_element_type=jnp.float32) + b2_ref[...]
    ze_ref[...] = (obs_ref[...][:, None, :] * encwT_ref[...][None, :, :]
                   + encbT_ref[...][None, :, :])


def _sc_gather(table, fidx2):
    n_idx = fidx2.shape[1]
    window = 128

    @pl.kernel(
        out_type=jax.ShapeDtypeStruct((n_idx, 2 * CODE_DIM), table.dtype),
        mesh=plsc.VectorSubcoreMesh(core_axis_name="core",
                                    subcore_axis_name="subcore"),
    )
    def kern(x_hbm, i_hbm, o_hbm):
        def body(i_vmem, o_vmem):
            pltpu.sync_copy(x_hbm.at[i_vmem.at[0]], o_vmem)

        pltpu.emit_pipeline(
            body,
            grid=(n_idx // window,),
            in_specs=[pl.BlockSpec((1, window), index_map=lambda i: (0, i))],
            out_specs=[pl.BlockSpec((window, 2 * CODE_DIM),
                                    index_map=lambda i: (i, 0))],
            core_axis_name=("core", "subcore"),
            dimension_semantics=(pltpu.PARALLEL,),
        )(i_hbm, o_hbm)

    return kern(table, fidx2)


def kernel(obs, enc_w, enc_b, emb_weight, dec_w1, dec_b1, dec_w2, dec_b2):
    obsT = obs.T
    encwT = enc_w.T
    encbT = enc_b.T

    fidx2T, parT, pair_table = pl.pallas_call(
        _argmin_body,
        grid=(OBS_DIM,),
        in_specs=[
            pl.BlockSpec((CODE_DIM, N_CODE_EACH), lambda j: (0, j)),
            pl.BlockSpec((OBS_DIM, BATCH), lambda j: (0, 0)),
            pl.BlockSpec((CODE_DIM, OBS_DIM), lambda j: (0, 0)),
            pl.BlockSpec((CODE_DIM, OBS_DIM), lambda j: (0, 0)),
        ],
        out_specs=[
            pl.BlockSpec((OBS_DIM, BATCH), lambda j: (0, 0)),
            pl.BlockSpec((OBS_DIM, BATCH), lambda j: (0, 0)),
            pl.BlockSpec((N_CODE_EACH // 2, 2 * CODE_DIM), lambda j: (j, 0)),
        ],
        out_shape=[
            jax.ShapeDtypeStruct((OBS_DIM, BATCH), jnp.int32),
            jax.ShapeDtypeStruct((OBS_DIM, BATCH), jnp.int32),
            jax.ShapeDtypeStruct((N_CODE_TOTAL // 2, 2 * CODE_DIM),
                                 jnp.float32),
        ],
    )(emb_weight, obsT, encwT, encbT)

    fidx2 = fidx2T.T.reshape(1, BATCH * OBS_DIM)
    par2 = parT.T
    q2 = _sc_gather(pair_table, fidx2)
    q2v = q2.reshape(BATCH, OBS_DIM, 2 * CODE_DIM)

    bb = 128
    recon, ze, emb = pl.pallas_call(
        _decoder_body,
        grid=(BATCH // bb,),
        in_specs=[
            pl.BlockSpec((bb, OBS_DIM, 2 * CODE_DIM), lambda i: (i, 0, 0)),
            pl.BlockSpec((bb, OBS_DIM), lambda i: (i, 0)),
            pl.BlockSpec((bb, OBS_DIM), lambda i: (i, 0)),
            pl.BlockSpec((CODE_DIM, OBS_DIM), lambda i: (0, 0)),
            pl.BlockSpec((CODE_DIM, OBS_DIM), lambda i: (0, 0)),
            pl.BlockSpec((REP_DIM, HIDDEN), lambda i: (0, 0)),
            pl.BlockSpec((1, HIDDEN), lambda i: (0, 0)),
            pl.BlockSpec((HIDDEN, OBS_DIM), lambda i: (0, 0)),
            pl.BlockSpec((1, OBS_DIM), lambda i: (0, 0)),
        ],
        out_specs=[
            pl.BlockSpec((bb, OBS_DIM), lambda i: (i, 0)),
            pl.BlockSpec((bb, CODE_DIM, OBS_DIM), lambda i: (i, 0, 0)),
            pl.BlockSpec((bb, CODE_DIM, OBS_DIM), lambda i: (i, 0, 0)),
        ],
        out_shape=[
            jax.ShapeDtypeStruct((BATCH, OBS_DIM), jnp.float32),
            jax.ShapeDtypeStruct((BATCH, CODE_DIM, OBS_DIM), jnp.float32),
            jax.ShapeDtypeStruct((BATCH, CODE_DIM, OBS_DIM), jnp.float32),
        ],
    )(q2v, par2, obs, encwT, encbT, dec_w1, dec_b1.reshape(1, HIDDEN),
      dec_w2, dec_b2.reshape(1, OBS_DIM))

    return recon, ze, emb


# speedup vs baseline: 1.6269x; 1.1687x over previous
import jax
import jax.numpy as jnp
from jax.experimental import pallas as pl
from jax.experimental.pallas import tpu as pltpu
from jax.experimental.pallas import tpu_sc as plsc

OBS_DIM = 64
N_CODE_EACH = 512
CODE_DIM = 64
BATCH = 1024
HIDDEN = 256
N_CODE_TOTAL = OBS_DIM * N_CODE_EACH
REP_DIM = OBS_DIM * CODE_DIM


def _argmin_body(emb_ref, obsT_ref, encwT_ref, encbT_ref,
                 fidx2_ref, par_ref, pair_ref):
    j = pl.program_id(0)
    Wt = emb_ref[...].T
    pair_ref[...] = jnp.concatenate(
        [Wt[:N_CODE_EACH // 2, :], Wt[N_CODE_EACH // 2:, :]], axis=1)
    ohj = jax.lax.broadcasted_iota(jnp.int32, (CODE_DIM, OBS_DIM), 1) == j
    ewc = jnp.sum(jnp.where(ohj, encwT_ref[...], 0.0), axis=1, keepdims=True)
    ebc = jnp.sum(jnp.where(ohj, encbT_ref[...], 0.0), axis=1, keepdims=True)
    ob = obsT_ref[pl.ds(j, 1), :]
    zT = ob * ewc + ebc
    cross = jnp.dot(Wt, zT, preferred_element_type=jnp.float32)
    z2 = jnp.sum(zT * zT, axis=0, keepdims=True)
    w2 = jnp.sum(Wt * Wt, axis=1, keepdims=True)
    dists = (z2 + w2) - 2.0 * cross
    m = jnp.min(dists, axis=0, keepdims=True)
    kio = jax.lax.broadcasted_iota(jnp.int32, dists.shape, 0)
    cand = jnp.where(dists == m, kio, N_CODE_EACH)
    idx = jnp.min(cand, axis=0, keepdims=True)
    fidx2_ref[pl.ds(j, 1), :] = (idx & (N_CODE_EACH // 2 - 1)) + j * (
        N_CODE_EACH // 2)
    par_ref[pl.ds(j, 1), :] = idx >> 8


def _decoder_body(q2_ref, par_ref, obsT_ref, encwT_ref, encbT_ref,
                  w1_ref, b1_ref, w2_ref, b2_ref,
                  recon_ref, ze_ref, emb_ref):
    q2 = q2_ref[...]
    par3 = par_ref[...][:, :, None]
    sel = jnp.where(par3 == 0,
                    q2[:, :, :CODE_DIM], q2[:, :, CODE_DIM:])
    emb_ref[...] = sel
    qf = sel.reshape(sel.shape[0], REP_DIM)
    h = jnp.dot(qf, w1_ref[...],
                preferred_element_type=jnp.float32) + b1_ref[...]
    h = jnp.maximum(h, 0.0)
    recon_ref[...] = jnp.dot(h, w2_ref[...],
                             preferred_element_type=jnp.float32) + b2_ref[...]
    ze3 = (obsT_ref[...][None, :, :] * encwT_ref[...][:, :, None]
           + encbT_ref[...][:, :, None])
    ze_ref[...] = ze3.reshape(REP_DIM, ze3.shape[2])


def _sc_gather(table, fidx2):
    n_idx = fidx2.shape[1]
    window = 128

    @pl.kernel(
        out_type=jax.ShapeDtypeStruct((n_idx, 2 * CODE_DIM), table.dtype),
        mesh=plsc.VectorSubcoreMesh(core_axis_name="core",
                                    subcore_axis_name="subcore"),
    )
    def kern(x_hbm, i_hbm, o_hbm):
        def body(i_vmem, o_vmem):
            pltpu.sync_copy(x_hbm.at[i_vmem.at[0]], o_vmem)

        pltpu.emit_pipeline(
            body,
            grid=(n_idx // window,),
            in_specs=[pl.BlockSpec((1, window), index_map=lambda i: (0, i))],
            out_specs=[pl.BlockSpec((window, 2 * CODE_DIM),
                                    index_map=lambda i: (i, 0))],
            core_axis_name=("core", "subcore"),
            dimension_semantics=(pltpu.PARALLEL,),
        )(i_hbm, o_hbm)

    return kern(table, fidx2)


def kernel(obs, enc_w, enc_b, emb_weight, dec_w1, dec_b1, dec_w2, dec_b2):
    obsT = obs.T
    encwT = enc_w.T
    encbT = enc_b.T

    fidx2T, parT, pair_table = pl.pallas_call(
        _argmin_body,
        grid=(OBS_DIM,),
        in_specs=[
            pl.BlockSpec((CODE_DIM, N_CODE_EACH), lambda j: (0, j)),
            pl.BlockSpec((OBS_DIM, BATCH), lambda j: (0, 0)),
            pl.BlockSpec((CODE_DIM, OBS_DIM), lambda j: (0, 0)),
            pl.BlockSpec((CODE_DIM, OBS_DIM), lambda j: (0, 0)),
        ],
        out_specs=[
            pl.BlockSpec((OBS_DIM, BATCH), lambda j: (0, 0)),
            pl.BlockSpec((OBS_DIM, BATCH), lambda j: (0, 0)),
            pl.BlockSpec((N_CODE_EACH // 2, 2 * CODE_DIM), lambda j: (j, 0)),
        ],
        out_shape=[
            jax.ShapeDtypeStruct((OBS_DIM, BATCH), jnp.int32),
            jax.ShapeDtypeStruct((OBS_DIM, BATCH), jnp.int32),
            jax.ShapeDtypeStruct((N_CODE_TOTAL // 2, 2 * CODE_DIM),
                                 jnp.float32),
        ],
    )(emb_weight, obsT, encwT, encbT)

    fidx2 = fidx2T.T.reshape(1, BATCH * OBS_DIM)
    par2 = parT.T
    q2 = _sc_gather(pair_table, fidx2)
    q2v = q2.reshape(BATCH, OBS_DIM, 2 * CODE_DIM)

    bb = 128
    recon, ze2, embJD = pl.pallas_call(
        _decoder_body,
        grid=(BATCH // bb,),
        in_specs=[
            pl.BlockSpec((bb, OBS_DIM, 2 * CODE_DIM), lambda i: (i, 0, 0)),
            pl.BlockSpec((bb, OBS_DIM), lambda i: (i, 0)),
            pl.BlockSpec((OBS_DIM, bb), lambda i: (0, i)),
            pl.BlockSpec((CODE_DIM, OBS_DIM), lambda i: (0, 0)),
            pl.BlockSpec((CODE_DIM, OBS_DIM), lambda i: (0, 0)),
            pl.BlockSpec((REP_DIM, HIDDEN), lambda i: (0, 0)),
            pl.BlockSpec((1, HIDDEN), lambda i: (0, 0)),
            pl.BlockSpec((HIDDEN, OBS_DIM), lambda i: (0, 0)),
            pl.BlockSpec((1, OBS_DIM), lambda i: (0, 0)),
        ],
        out_specs=[
            pl.BlockSpec((bb, OBS_DIM), lambda i: (i, 0)),
            pl.BlockSpec((REP_DIM, bb), lambda i: (0, i)),
            pl.BlockSpec((bb, OBS_DIM, CODE_DIM), lambda i: (i, 0, 0)),
        ],
        out_shape=[
            jax.ShapeDtypeStruct((BATCH, OBS_DIM), jnp.float32),
            jax.ShapeDtypeStruct((REP_DIM, BATCH), jnp.float32),
            jax.ShapeDtypeStruct((BATCH, OBS_DIM, CODE_DIM), jnp.float32),
        ],
    )(q2v, par2, obsT, encwT, encbT, dec_w1, dec_b1.reshape(1, HIDDEN),
      dec_w2, dec_b2.reshape(1, OBS_DIM))

    ze = jnp.transpose(ze2.reshape(CODE_DIM, OBS_DIM, BATCH), (2, 0, 1))
    emb = jnp.swapaxes(embJD, 1, 2)
    return recon, ze, emb


# speedup vs baseline: 1.6591x; 1.0198x over previous
import functools

import jax
import jax.numpy as jnp
from jax.experimental import pallas as pl
from jax.experimental.pallas import tpu as pltpu
from jax.experimental.pallas import tpu_sc as plsc

OBS_DIM = 64
N_CODE_EACH = 512
CODE_DIM = 64
BATCH = 1024
HIDDEN = 256
N_CODE_TOTAL = OBS_DIM * N_CODE_EACH
REP_DIM = OBS_DIM * CODE_DIM

J_CHUNKS = 2
JC = OBS_DIM // J_CHUNKS
KH = N_CODE_EACH // 2


def _argmin_body(j0, emb_ref, obsT_ref, encwT_ref, encbT_ref,
                 fidx2_ref, par_ref, pair_ref):
    j = pl.program_id(0)
    jg = j + j0
    Wt = emb_ref[...].T
    pair_ref[...] = jnp.concatenate([Wt[:KH, :], Wt[KH:, :]], axis=1)
    ohj = jax.lax.broadcasted_iota(jnp.int32, (CODE_DIM, OBS_DIM), 1) == jg
    ewc = jnp.sum(jnp.where(ohj, encwT_ref[...], 0.0), axis=1, keepdims=True)
    ebc = jnp.sum(jnp.where(ohj, encbT_ref[...], 0.0), axis=1, keepdims=True)
    ob = obsT_ref[pl.ds(jg, 1), :]
    zT = ob * ewc + ebc
    cross = jnp.dot(Wt, zT, preferred_element_type=jnp.float32)
    z2 = jnp.sum(zT * zT, axis=0, keepdims=True)
    w2 = jnp.sum(Wt * Wt, axis=1, keepdims=True)
    dists = (z2 + w2) - 2.0 * cross
    m = jnp.min(dists, axis=0, keepdims=True)
    kio = jax.lax.broadcasted_iota(jnp.int32, dists.shape, 0)
    cand = jnp.where(dists == m, kio, N_CODE_EACH)
    idx = jnp.min(cand, axis=0, keepdims=True)
    fidx2_ref[pl.ds(j, 1), :] = (idx & (KH - 1)) + j * KH
    par_ref[pl.ds(j, 1), :] = idx >> 8


def _ze_body(obsT_ref, encwT_ref, encbT_ref, ze_ref):
    ze3 = (obsT_ref[...][None, :, :] * encwT_ref[...][:, :, None]
           + encbT_ref[...][:, :, None])
    ze_ref[...] = ze3.reshape(REP_DIM, ze3.shape[2])


def _decoder_body(q2a_ref, q2b_ref, para_ref, parb_ref,
                  w1_ref, b1_ref, w2_ref, b2_ref,
                  recon_ref, emb_ref):
    sels = []
    for q2_ref, par_ref in ((q2a_ref, para_ref), (q2b_ref, parb_ref)):
        q2 = q2_ref[...]
        par3 = par_ref[...][:, :, None]
        sels.append(jnp.where(par3 == 0,
                              q2[:, :, :CODE_DIM], q2[:, :, CODE_DIM:]))
    sel = jnp.concatenate(sels, axis=1)
    emb_ref[...] = sel
    qf = sel.reshape(sel.shape[0], REP_DIM)
    h = jnp.dot(qf, w1_ref[...],
                preferred_element_type=jnp.float32) + b1_ref[...]
    h = jnp.maximum(h, 0.0)
    recon_ref[...] = jnp.dot(h, w2_ref[...],
                             preferred_element_type=jnp.float32) + b2_ref[...]


def _sc_gather(table, fidx2):
    n_idx = fidx2.shape[1]
    window = 128

    @pl.kernel(
        out_type=jax.ShapeDtypeStruct((n_idx, 2 * CODE_DIM), table.dtype),
        mesh=plsc.VectorSubcoreMesh(core_axis_name="core",
                                    subcore_axis_name="subcore"),
    )
    def kern(x_hbm, i_hbm, o_hbm):
        def body(i_vmem, o_vmem):
            pltpu.sync_copy(x_hbm.at[i_vmem.at[0]], o_vmem)

        pltpu.emit_pipeline(
            body,
            grid=(n_idx // window,),
            in_specs=[pl.BlockSpec((1, window), index_map=lambda i: (0, i))],
            out_specs=[pl.BlockSpec((window, 2 * CODE_DIM),
                                    index_map=lambda i: (i, 0))],
            core_axis_name=("core", "subcore"),
            dimension_semantics=(pltpu.PARALLEL,),
        )(i_hbm, o_hbm)

    return kern(table, fidx2)


def _argmin_chunk(c, emb_weight, obsT, encwT, encbT):
    return pl.pallas_call(
        functools.partial(_argmin_body, c * JC),
        grid=(JC,),
        in_specs=[
            pl.BlockSpec((CODE_DIM, N_CODE_EACH),
                         lambda j, c=c: (0, c * JC + j)),
            pl.BlockSpec((OBS_DIM, BATCH), lambda j: (0, 0)),
            pl.BlockSpec((CODE_DIM, OBS_DIM), lambda j: (0, 0)),
            pl.BlockSpec((CODE_DIM, OBS_DIM), lambda j: (0, 0)),
        ],
        out_specs=[
            pl.BlockSpec((JC, BATCH), lambda j: (0, 0)),
            pl.BlockSpec((JC, BATCH), lambda j: (0, 0)),
            pl.BlockSpec((KH, 2 * CODE_DIM), lambda j: (j, 0)),
        ],
        out_shape=[
            jax.ShapeDtypeStruct((JC, BATCH), jnp.int32),
            jax.ShapeDtypeStruct((JC, BATCH), jnp.int32),
            jax.ShapeDtypeStruct((JC * KH, 2 * CODE_DIM), jnp.float32),
        ],
    )(emb_weight, obsT, encwT, encbT)


def kernel(obs, enc_w, enc_b, emb_weight, dec_w1, dec_b1, dec_w2, dec_b2):
    obsT = obs.T
    encwT = enc_w.T
    encbT = enc_b.T

    q2vs, pars = [], []
    for c in range(J_CHUNKS):
        fidx2T_c, parT_c, pair_c = _argmin_chunk(
            c, emb_weight, obsT, encwT, encbT)
        fidx2_c = fidx2T_c.T.reshape(1, BATCH * JC)
        q2_c = _sc_gather(pair_c, fidx2_c)
        q2vs.append(q2_c.reshape(BATCH, JC, 2 * CODE_DIM))
        pars.append(parT_c.T)

    ze2 = pl.pallas_call(
        _ze_body,
        grid=(BATCH // 256,),
        in_specs=[
            pl.BlockSpec((OBS_DIM, 256), lambda i: (0, i)),
            pl.BlockSpec((CODE_DIM, OBS_DIM), lambda i: (0, 0)),
            pl.BlockSpec((CODE_DIM, OBS_DIM), lambda i: (0, 0)),
        ],
        out_specs=pl.BlockSpec((REP_DIM, 256), lambda i: (0, i)),
        out_shape=jax.ShapeDtypeStruct((REP_DIM, BATCH), jnp.float32),
    )(obsT, encwT, encbT)

    bb = 128
    recon, embJD = pl.pallas_call(
        _decoder_body,
        grid=(BATCH // bb,),
        in_specs=[
            pl.BlockSpec((bb, JC, 2 * CODE_DIM), lambda i: (i, 0, 0)),
            pl.BlockSpec((bb, JC, 2 * CODE_DIM), lambda i: (i, 0, 0)),
            pl.BlockSpec((bb, JC), lambda i: (i, 0)),
            pl.BlockSpec((bb, JC), lambda i: (i, 0)),
            pl.BlockSpec((REP_DIM, HIDDEN), lambda i: (0, 0)),
            pl.BlockSpec((1, HIDDEN), lambda i: (0, 0)),
            pl.BlockSpec((HIDDEN, OBS_DIM), lambda i: (0, 0)),
            pl.BlockSpec((1, OBS_DIM), lambda i: (0, 0)),
        ],
        out_specs=[
            pl.BlockSpec((bb, OBS_DIM), lambda i: (i, 0)),
            pl.BlockSpec((bb, OBS_DIM, CODE_DIM), lambda i: (i, 0, 0)),
        ],
        out_shape=[
            jax.ShapeDtypeStruct((BATCH, OBS_DIM), jnp.float32),
            jax.ShapeDtypeStruct((BATCH, OBS_DIM, CODE_DIM), jnp.float32),
        ],
    )(q2vs[0], q2vs[1], pars[0], pars[1], dec_w1,
      dec_b1.reshape(1, HIDDEN), dec_w2, dec_b2.reshape(1, OBS_DIM))

    ze = jnp.transpose(ze2.reshape(CODE_DIM, OBS_DIM, BATCH), (2, 0, 1))
    emb = jnp.swapaxes(embJD, 1, 2)
    return recon, ze, emb


# speedup vs baseline: 1.6844x; 1.0153x over previous
import functools

import jax
import jax.numpy as jnp
from jax.experimental import pallas as pl
from jax.experimental.pallas import tpu as pltpu
from jax.experimental.pallas import tpu_sc as plsc

OBS_DIM = 64
N_CODE_EACH = 512
CODE_DIM = 64
BATCH = 1024
HIDDEN = 256
N_CODE_TOTAL = OBS_DIM * N_CODE_EACH
REP_DIM = OBS_DIM * CODE_DIM

J_CHUNKS = 2
JC = OBS_DIM // J_CHUNKS
KH = N_CODE_EACH // 2


def _argmin_body(j0, emb_ref, obsT_ref, encwT_ref, encbT_ref,
                 fidx2_ref, par_ref, pair_ref):
    j = pl.program_id(0)
    jg = j + j0
    Wt = emb_ref[...].T
    pair_ref[...] = jnp.concatenate([Wt[:KH, :], Wt[KH:, :]], axis=1)
    ohj = jax.lax.broadcasted_iota(jnp.int32, (CODE_DIM, OBS_DIM), 1) == jg
    ewc = jnp.sum(jnp.where(ohj, encwT_ref[...], 0.0), axis=1, keepdims=True)
    ebc = jnp.sum(jnp.where(ohj, encbT_ref[...], 0.0), axis=1, keepdims=True)
    ob = obsT_ref[pl.ds(jg, 1), :]
    zT = ob * ewc + ebc
    cross = jnp.dot(Wt, zT, preferred_element_type=jnp.float32)
    z2 = jnp.sum(zT * zT, axis=0, keepdims=True)
    w2 = jnp.sum(Wt * Wt, axis=1, keepdims=True)
    dists = (z2 + w2) - 2.0 * cross
    m = jnp.min(dists, axis=0, keepdims=True)
    kio = jax.lax.broadcasted_iota(jnp.int32, dists.shape, 0)
    cand = jnp.where(dists == m, kio, N_CODE_EACH)
    idx = jnp.min(cand, axis=0, keepdims=True)
    fidx2_ref[pl.ds(j, 1), :] = (idx & (KH - 1)) + j * KH
    par_ref[pl.ds(j, 1), :] = idx >> 8


def _ze_body(obsT_ref, encwT_ref, encbT_ref, ze_ref):
    ze3 = (obsT_ref[...][None, :, :] * encwT_ref[...][:, :, None]
           + encbT_ref[...][:, :, None])
    ze_ref[...] = ze3.reshape(8 * OBS_DIM, BATCH)


def _decoder_body(q2a_ref, q2b_ref, para_ref, parb_ref,
                  w1_ref, b1_ref, w2_ref, b2_ref,
                  recon_ref, emb_ref):
    sels = []
    for q2_ref, par_ref in ((q2a_ref, para_ref), (q2b_ref, parb_ref)):
        q2 = q2_ref[...]
        par3 = par_ref[...][:, :, None]
        sels.append(jnp.where(par3 == 0,
                              q2[:, :, :CODE_DIM], q2[:, :, CODE_DIM:]))
    sel = jnp.concatenate(sels, axis=1)
    emb_ref[...] = sel
    qf = sel.reshape(sel.shape[0], REP_DIM)
    h = jnp.dot(qf, w1_ref[...],
                preferred_element_type=jnp.float32) + b1_ref[...]
    h = jnp.maximum(h, 0.0)
    recon_ref[...] = jnp.dot(h, w2_ref[...],
                             preferred_element_type=jnp.float32) + b2_ref[...]


def _sc_gather(table, fidx2):
    n_idx = fidx2.shape[1]
    window = 256

    @pl.kernel(
        out_type=jax.ShapeDtypeStruct((n_idx, 2 * CODE_DIM), table.dtype),
        mesh=plsc.VectorSubcoreMesh(core_axis_name="core",
                                    subcore_axis_name="subcore"),
    )
    def kern(x_hbm, i_hbm, o_hbm):
        def body(i_vmem, o_vmem):
            pltpu.sync_copy(x_hbm.at[i_vmem.at[0]], o_vmem)

        pltpu.emit_pipeline(
            body,
            grid=(n_idx // window,),
            in_specs=[pl.BlockSpec((1, window), index_map=lambda i: (0, i))],
            out_specs=[pl.BlockSpec((window, 2 * CODE_DIM),
                                    index_map=lambda i: (i, 0))],
            core_axis_name=("core", "subcore"),
            dimension_semantics=(pltpu.PARALLEL,),
        )(i_hbm, o_hbm)

    return kern(table, fidx2)


def _argmin_chunk(c, emb_weight, obsT, encwT, encbT):
    return pl.pallas_call(
        functools.partial(_argmin_body, c * JC),
        grid=(JC,),
        in_specs=[
            pl.BlockSpec((CODE_DIM, N_CODE_EACH),
                         lambda j, c=c: (0, c * JC + j)),
            pl.BlockSpec((OBS_DIM, BATCH), lambda j: (0, 0)),
            pl.BlockSpec((CODE_DIM, OBS_DIM), lambda j: (0, 0)),
            pl.BlockSpec((CODE_DIM, OBS_DIM), lambda j: (0, 0)),
        ],
        out_specs=[
            pl.BlockSpec((JC, BATCH), lambda j: (0, 0)),
            pl.BlockSpec((JC, BATCH), lambda j: (0, 0)),
            pl.BlockSpec((KH, 2 * CODE_DIM), lambda j: (j, 0)),
        ],
        out_shape=[
            jax.ShapeDtypeStruct((JC, BATCH), jnp.int32),
            jax.ShapeDtypeStruct((JC, BATCH), jnp.int32),
            jax.ShapeDtypeStruct((JC * KH, 2 * CODE_DIM), jnp.float32),
        ],
    )(emb_weight, obsT, encwT, encbT)


def kernel(obs, enc_w, enc_b, emb_weight, dec_w1, dec_b1, dec_w2, dec_b2):
    obsT = obs.T
    encwT = enc_w.T
    encbT = enc_b.T

    q2vs, pars = [], []
    for c in range(J_CHUNKS):
        fidx2T_c, parT_c, pair_c = _argmin_chunk(
            c, emb_weight, obsT, encwT, encbT)
        fidx2_c = fidx2T_c.T.reshape(1, BATCH * JC)
        q2_c = _sc_gather(pair_c, fidx2_c)
        q2vs.append(q2_c.reshape(BATCH, JC, 2 * CODE_DIM))
        pars.append(parT_c.T)

    ze2 = pl.pallas_call(
        _ze_body,
        grid=(CODE_DIM // 8,),
        in_specs=[
            pl.BlockSpec((OBS_DIM, BATCH), lambda i: (0, 0)),
            pl.BlockSpec((8, OBS_DIM), lambda i: (i, 0)),
            pl.BlockSpec((8, OBS_DIM), lambda i: (i, 0)),
        ],
        out_specs=pl.BlockSpec((8 * OBS_DIM, BATCH), lambda i: (i, 0)),
        out_shape=jax.ShapeDtypeStruct((REP_DIM, BATCH), jnp.float32),
    )(obsT, encwT, encbT)

    bb = 128
    recon, embJD = pl.pallas_call(
        _decoder_body,
        grid=(BATCH // bb,),
        in_specs=[
            pl.BlockSpec((bb, JC, 2 * CODE_DIM), lambda i: (i, 0, 0)),
            pl.BlockSpec((bb, JC, 2 * CODE_DIM), lambda i: (i, 0, 0)),
            pl.BlockSpec((bb, JC), lambda i: (i, 0)),
            pl.BlockSpec((bb, JC), lambda i: (i, 0)),
            pl.BlockSpec((REP_DIM, HIDDEN), lambda i: (0, 0)),
            pl.BlockSpec((1, HIDDEN), lambda i: (0, 0)),
            pl.BlockSpec((HIDDEN, OBS_DIM), lambda i: (0, 0)),
            pl.BlockSpec((1, OBS_DIM), lambda i: (0, 0)),
        ],
        out_specs=[
            pl.BlockSpec((bb, OBS_DIM), lambda i: (i, 0)),
            pl.BlockSpec((bb, OBS_DIM, CODE_DIM), lambda i: (i, 0, 0)),
        ],
        out_shape=[
            jax.ShapeDtypeStruct((BATCH, OBS_DIM), jnp.float32),
            jax.ShapeDtypeStruct((BATCH, OBS_DIM, CODE_DIM), jnp.float32),
        ],
    )(q2vs[0], q2vs[1], pars[0], pars[1], dec_w1,
      dec_b1.reshape(1, HIDDEN), dec_w2, dec_b2.reshape(1, OBS_DIM))

    ze = jnp.transpose(ze2.reshape(CODE_DIM, OBS_DIM, BATCH), (2, 0, 1))
    emb = jnp.swapaxes(embJD, 1, 2)
    return recon, ze, emb
